# baseline (speedup 1.0000x reference)
import functools

import jax
import jax.numpy as jnp
from jax import lax
from jax.experimental import pallas as pl
from jax.experimental.pallas import tpu as pltpu
from jax.experimental.pallas import tpu_sc as plsc

SC_B = 1


def _sc_part(inputs, table):
    B, S, D = inputs.shape
    info = plsc.get_sparse_core_info()
    NC, NS, L = info.num_cores, info.num_subcores, info.num_lanes
    NW = NC * NS
    SPW = S // NW
    CS = 32
    NCHUNK = SPW // CS
    NVC = D // L

    mesh = plsc.VectorSubcoreMesh(core_axis_name="c", subcore_axis_name="s")

    @functools.partial(
        pl.kernel,
        mesh=mesh,
        out_type=jax.ShapeDtypeStruct((SC_B, S, D), jnp.float32),
        scratch_types=[
            pltpu.VMEM((CS, D), jnp.float32),
            pltpu.VMEM((CS, D), jnp.float32),
        ],
    )
    def k(x_hbm, t_hbm, o_hbm, tbuf, xbuf):
        wid = lax.axis_index("s") * NC + lax.axis_index("c")
        base = wid * SPW

        def chunk_body(c, carry):
            s0 = base + c * CS
            pltpu.sync_copy(t_hbm.at[pl.ds(s0, CS)], tbuf)
            for b in range(SC_B):
                pltpu.sync_copy(x_hbm.at[b, pl.ds(s0, CS)], xbuf)

                def row_body(r, carry2):
                    for cc in range(NVC):
                        sl = pl.ds(cc * L, L)
                        xbuf[r, sl] = xbuf[r, sl] + tbuf[r, sl]
                    return carry2

                lax.fori_loop(0, CS, row_body, 0)
                pltpu.sync_copy(xbuf, o_hbm.at[b, pl.ds(s0, CS)])
            return carry

        lax.fori_loop(0, NCHUNK, chunk_body, 0)

    return k(inputs, table)


def _tc_part(inputs, table):
    B, S, D = inputs.shape
    BS = 2048
    TB = B - SC_B

    def body(x_ref, t_ref, o_ref):
        o_ref[...] = x_ref[...] + t_ref[...]

    return pl.pallas_call(
        body,
        grid=(S // BS, TB),
        in_specs=[
            pl.BlockSpec((1, BS, D), lambda s, b: (b + SC_B, s, 0)),
            pl.BlockSpec((BS, D), lambda s, b: (s, 0)),
        ],
        out_specs=pl.BlockSpec((1, BS, D), lambda s, b: (b, s, 0)),
        out_shape=jax.ShapeDtypeStruct((TB, S, D), inputs.dtype),
    )(inputs, table)


def kernel(inputs, table):
    sc_out = _sc_part(inputs, table)
    tc_out = _tc_part(inputs, table)
    return jnp.concatenate([sc_out, tc_out], axis=0)


# speedup vs baseline: 1.5403x; 1.5403x over previous
import functools

import jax
import jax.numpy as jnp
from jax import lax
from jax.experimental import pallas as pl
from jax.experimental.pallas import tpu as pltpu
from jax.experimental.pallas import tpu_sc as plsc

S_SC = 2048
BS = 2048


def _sc_part(inputs, table):
    B, S, D = inputs.shape
    info = plsc.get_sparse_core_info()
    NC, NS, L = info.num_cores, info.num_subcores, info.num_lanes
    NW = NC * NS
    SPW = S_SC // NW
    CS = 32
    NCHUNK = SPW // CS
    NVC = D // L

    mesh = plsc.VectorSubcoreMesh(core_axis_name="c", subcore_axis_name="s")

    @functools.partial(
        pl.kernel,
        mesh=mesh,
        out_type=jax.ShapeDtypeStruct((B, S_SC, D), jnp.float32),
        scratch_types=[
            pltpu.VMEM((CS, D), jnp.float32),
            pltpu.VMEM((CS, D), jnp.float32),
        ],
    )
    def k(x_hbm, t_hbm, o_hbm, tbuf, xbuf):
        wid = lax.axis_index("s") * NC + lax.axis_index("c")
        base = wid * SPW

        def chunk_body(c, carry):
            s0 = base + c * CS
            pltpu.sync_copy(t_hbm.at[pl.ds(s0, CS)], tbuf)
            for b in range(B):
                pltpu.sync_copy(x_hbm.at[b, pl.ds(s0, CS)], xbuf)

                def row_body(r, carry2):
                    for cc in range(NVC):
                        sl = pl.ds(cc * L, L)
                        xbuf[r, sl] = xbuf[r, sl] + tbuf[r, sl]
                    return carry2

                lax.fori_loop(0, CS, row_body, 0)
                pltpu.sync_copy(xbuf, o_hbm.at[b, pl.ds(s0, CS)])
            return carry

        lax.fori_loop(0, NCHUNK, chunk_body, 0)

    return k(inputs, table)


def _tc_part(inputs, table):
    B, S, D = inputs.shape
    OFF = S_SC // BS

    def body(x_ref, t_ref, o_ref):
        o_ref[...] = x_ref[...] + t_ref[...]

    return pl.pallas_call(
        body,
        grid=((S - S_SC) // BS, B),
        in_specs=[
            pl.BlockSpec((1, BS, D), lambda s, b: (b, s + OFF, 0)),
            pl.BlockSpec((BS, D), lambda s, b: (s + OFF, 0)),
        ],
        out_specs=pl.BlockSpec((1, BS, D), lambda s, b: (b, s + OFF, 0)),
        out_shape=jax.ShapeDtypeStruct((B, S, D), inputs.dtype),
    )(inputs, table)


def kernel(inputs, table):
    sc_out = _sc_part(inputs, table)
    tc_full = _tc_part(inputs, table)
    return lax.dynamic_update_slice(tc_full, sc_out, (0, 0, 0))


# speedup vs baseline: 1.7130x; 1.1121x over previous
import functools

import jax
import jax.numpy as jnp
from jax import lax
from jax.experimental import pallas as pl
from jax.experimental.pallas import tpu as pltpu
from jax.experimental.pallas import tpu_sc as plsc

S_SC = 2048
BS = 2048


def _sc_part(inputs, table):
    B, S, D = inputs.shape
    info = plsc.get_sparse_core_info()
    NC, NS, L = info.num_cores, info.num_subcores, info.num_lanes
    NW = NC * NS
    SPW = S_SC // NW
    CS = 32
    NCHUNK = SPW // CS
    NVC = D // L

    mesh = plsc.VectorSubcoreMesh(core_axis_name="c", subcore_axis_name="s")

    @functools.partial(
        pl.kernel,
        mesh=mesh,
        out_type=jax.ShapeDtypeStruct((1, S_SC, D), jnp.float32),
        scratch_types=[
            pltpu.VMEM((CS, D), jnp.float32),
            pltpu.VMEM((CS, D), jnp.float32),
        ],
    )
    def k(x_hbm, t_hbm, o_hbm, tbuf, xbuf):
        wid = lax.axis_index("s") * NC + lax.axis_index("c")
        base = wid * SPW

        def chunk_body(c, carry):
            s0 = base + c * CS
            pltpu.sync_copy(t_hbm.at[pl.ds(s0, CS)], tbuf)
            pltpu.sync_copy(x_hbm.at[0, pl.ds(s0, CS)], xbuf)

            def row_body(r, carry2):
                for cc in range(NVC):
                    sl = pl.ds(cc * L, L)
                    xbuf[r, sl] = xbuf[r, sl] + tbuf[r, sl]
                return carry2

            lax.fori_loop(0, CS, row_body, 0)
            pltpu.sync_copy(xbuf, o_hbm.at[0, pl.ds(s0, CS)])
            return carry

        lax.fori_loop(0, NCHUNK, chunk_body, 0)

    return k(inputs, table)


def _tc_part(inputs, table):
    B, S, D = inputs.shape
    NSB = S // BS
    NBLK = B * NSB - S_SC // BS

    def body(x_ref, t_ref, o_ref):
        o_ref[...] = x_ref[...] + t_ref[...]

    skip = S_SC // BS

    return pl.pallas_call(
        body,
        grid=(NBLK,),
        in_specs=[
            pl.BlockSpec((1, BS, D), lambda i: ((i + skip) % B, (i + skip) // B, 0)),
            pl.BlockSpec((BS, D), lambda i: ((i + skip) // B, 0)),
        ],
        out_specs=pl.BlockSpec(
            (1, BS, D), lambda i: ((i + skip) % B, (i + skip) // B, 0)
        ),
        out_shape=jax.ShapeDtypeStruct((B, S, D), inputs.dtype),
    )(inputs, table)


def kernel(inputs, table):
    sc_out = _sc_part(inputs, table)
    tc_full = _tc_part(inputs, table)
    return lax.dynamic_update_slice(tc_full, sc_out, (0, 0, 0))


# speedup vs baseline: 2.2791x; 1.3305x over previous
import jax
import jax.numpy as jnp
from jax.experimental import pallas as pl


def kernel(inputs, table):
    B, S, D = inputs.shape
    BS = 2048

    def body(x_ref, t_ref, o_ref):
        o_ref[...] = x_ref[...] + t_ref[...]

    return pl.pallas_call(
        body,
        grid=(S // BS, B),
        in_specs=[
            pl.BlockSpec((1, BS, D), lambda s, b: (b, s, 0)),
            pl.BlockSpec((BS, D), lambda s, b: (s, 0)),
        ],
        out_specs=pl.BlockSpec((1, BS, D), lambda s, b: (b, s, 0)),
        out_shape=jax.ShapeDtypeStruct((B, S, D), inputs.dtype),
    )(inputs, table)
